# Initial kernel scaffold; baseline (speedup 1.0000x reference)
import jax
import jax.numpy as jnp
from jax.experimental import pallas as pl


def kernel(x, edge_index, W1, b1, g1, be1, W2, b2, g2, be2, Wh, bh):
    raise NotImplementedError("write your pallas kernel here")


# speedup vs baseline: 15.4849x; 15.4849x over previous
import functools

import jax
import jax.numpy as jnp
from jax import lax
from jax.experimental import pallas as pl
from jax.experimental.pallas import tpu as pltpu
from jax.experimental.pallas import tpu_sc as plsc

N = 10000
E = 320000
D = 128
EPS = 1e-5

NC = 2
NS = 16
NW = NC * NS
EPW = E // NW
CW = 125
CH = EPW // CW
RPT = 624
TAIL = N - NS * RPT

_mesh = plsc.VectorSubcoreMesh(
    core_axis_name="c", subcore_axis_name="s", num_cores=NC, num_subcores=NS)


@functools.partial(
    pl.kernel,
    out_type=jax.ShapeDtypeStruct((NW, N // 16, 16), jnp.float32),
    mesh=_mesh,
    compiler_params=pltpu.CompilerParams(needs_layout_passes=False),
    scratch_types=[
        pltpu.VMEM((EPW,), jnp.int32),
        pltpu.VMEM((N // 16, 16), jnp.float32),
    ],
)
def _deg_parts(col_hbm, out_hbm, colv, degv):
    cid = lax.axis_index("c")
    sid = lax.axis_index("s")
    wid = cid * NS + sid
    pltpu.sync_copy(col_hbm.at[pl.ds(wid * EPW, EPW)], colv)
    zero16 = jnp.zeros((16,), jnp.float32)
    one16 = jnp.ones((16,), jnp.float32)

    def zbody(i, carry):
        degv[i, :] = zero16
        return carry

    lax.fori_loop(0, N // 16, zbody, 0)

    def body(e, carry):
        idx = colv[pl.ds(e * 16, 16)]
        plsc.addupdate_scatter(degv, [idx >> 4, idx & 15], one16)
        return carry

    lax.fori_loop(0, EPW // 16, body, 0)
    pltpu.sync_copy(degv, out_hbm.at[wid])


@functools.partial(
    pl.kernel,
    out_type=jax.ShapeDtypeStruct((NC, N, D), jnp.float32),
    mesh=_mesh,
    scratch_types=[
        pltpu.VMEM((CH, CW), jnp.int32),
        pltpu.VMEM((CH, CW), jnp.int32),
        pltpu.VMEM((CW, D), jnp.float32),
        pltpu.VMEM_SHARED((N, D), jnp.float32),
        pltpu.SemaphoreType.DMA,
    ],
)
def _edge_scatter(hs_hbm, row_hbm, col_hbm, zeros_hbm, out_hbm,
                  rowv, colv, gbuf, acc_sh, sem):
    cid = lax.axis_index("c")
    sid = lax.axis_index("s")
    wid = cid * NS + sid
    pltpu.sync_copy(zeros_hbm.at[pl.ds(sid * RPT, RPT)],
                    acc_sh.at[pl.ds(sid * RPT, RPT)])

    @pl.when(sid == 0)
    def _():
        pltpu.sync_copy(zeros_hbm.at[pl.ds(NS * RPT, TAIL)],
                        acc_sh.at[pl.ds(NS * RPT, TAIL)])

    pltpu.sync_copy(row_hbm.at[pl.ds(wid * CH, CH)], rowv)
    pltpu.sync_copy(col_hbm.at[pl.ds(wid * CH, CH)], colv)
    plsc.subcore_barrier()

    def body(g, carry):
        pltpu.async_copy(hs_hbm.at[rowv.at[g]], gbuf, sem).wait()
        pltpu.sync_copy(gbuf, acc_sh.at[colv.at[g]], add=True)
        return carry

    lax.fori_loop(0, CH, body, 0)
    plsc.subcore_barrier()
    pltpu.sync_copy(acc_sh.at[pl.ds(sid * RPT, RPT)],
                    out_hbm.at[cid].at[pl.ds(sid * RPT, RPT)])

    @pl.when(sid == 0)
    def _():
        pltpu.sync_copy(acc_sh.at[pl.ds(NS * RPT, TAIL)],
                        out_hbm.at[cid].at[pl.ds(NS * RPT, TAIL)])


R = 1000
G = N // R

_HI = lax.Precision.HIGHEST


def _prep_body(x_ref, w_ref, dp_ref, hs_ref, dis_ref):
    deg = jnp.sum(dp_ref[...], axis=0) + 2.0
    dis = lax.rsqrt(deg)
    h = jnp.dot(x_ref[...], w_ref[...], precision=_HI,
                preferred_element_type=jnp.float32)
    hs_ref[...] = h * dis
    dis_ref[...] = dis


def _prep(x, W, deg_parts):
    return pl.pallas_call(
        _prep_body,
        grid=(G,),
        in_specs=[
            pl.BlockSpec((R, D), lambda i: (i, 0)),
            pl.BlockSpec((D, D), lambda i: (0, 0)),
            pl.BlockSpec((NW, R, 1), lambda i: (0, i, 0)),
        ],
        out_specs=[
            pl.BlockSpec((R, D), lambda i: (i, 0)),
            pl.BlockSpec((R, 1), lambda i: (i, 0)),
        ],
        out_shape=[
            jax.ShapeDtypeStruct((N, D), jnp.float32),
            jax.ShapeDtypeStruct((N, 1), jnp.float32),
        ],
    )(x, W, deg_parts)


def _agg_body(sp_ref, hs_ref, dis_ref, agg_ref, sum_ref, ssq_ref):
    i = pl.program_id(0)
    agg = dis_ref[...] * (sp_ref[0] + sp_ref[1] + 2.0 * hs_ref[...])
    agg_ref[...] = agg

    @pl.when(i == 0)
    def _():
        sum_ref[...] = jnp.zeros_like(sum_ref)
        ssq_ref[...] = jnp.zeros_like(ssq_ref)

    sum_ref[...] += jnp.sum(agg, axis=0, keepdims=True)
    ssq_ref[...] += jnp.sum(agg * agg, axis=0, keepdims=True)


def _agg(parts, hs, dis):
    return pl.pallas_call(
        _agg_body,
        grid=(G,),
        in_specs=[
            pl.BlockSpec((NC, R, D), lambda i: (0, i, 0)),
            pl.BlockSpec((R, D), lambda i: (i, 0)),
            pl.BlockSpec((R, 1), lambda i: (i, 0)),
        ],
        out_specs=[
            pl.BlockSpec((R, D), lambda i: (i, 0)),
            pl.BlockSpec((1, D), lambda i: (0, 0)),
            pl.BlockSpec((1, D), lambda i: (0, 0)),
        ],
        out_shape=[
            jax.ShapeDtypeStruct((N, D), jnp.float32),
            jax.ShapeDtypeStruct((1, D), jnp.float32),
            jax.ShapeDtypeStruct((1, D), jnp.float32),
        ],
    )(parts, hs, dis)


def _bn_relu_res(agg_ref, x_ref, sum_ref, ssq_ref, g_ref, be_ref):
    m = sum_ref[...] * (1.0 / N)
    v = ssq_ref[...] * (1.0 / N) - m * m
    scale = lax.rsqrt(v + EPS) * g_ref[...]
    y = (agg_ref[...] - m) * scale + be_ref[...]
    return jnp.maximum(y, 0.0) + x_ref[...]


def _bnmm_body(agg_ref, x_ref, sum_ref, ssq_ref, g_ref, be_ref, w_ref, dis_ref,
               x2_ref, hs2_ref):
    y = _bn_relu_res(agg_ref, x_ref, sum_ref, ssq_ref, g_ref, be_ref)
    x2_ref[...] = y
    h2 = jnp.dot(y, w_ref[...], precision=_HI, preferred_element_type=jnp.float32)
    hs2_ref[...] = h2 * dis_ref[...]


def _bnmm(agg, x, s, sq, g, be, W, dis):
    return pl.pallas_call(
        _bnmm_body,
        grid=(G,),
        in_specs=[
            pl.BlockSpec((R, D), lambda i: (i, 0)),
            pl.BlockSpec((R, D), lambda i: (i, 0)),
            pl.BlockSpec((1, D), lambda i: (0, 0)),
            pl.BlockSpec((1, D), lambda i: (0, 0)),
            pl.BlockSpec((1, D), lambda i: (0, 0)),
            pl.BlockSpec((1, D), lambda i: (0, 0)),
            pl.BlockSpec((D, D), lambda i: (0, 0)),
            pl.BlockSpec((R, 1), lambda i: (i, 0)),
        ],
        out_specs=[
            pl.BlockSpec((R, D), lambda i: (i, 0)),
            pl.BlockSpec((R, D), lambda i: (i, 0)),
        ],
        out_shape=[
            jax.ShapeDtypeStruct((N, D), jnp.float32),
            jax.ShapeDtypeStruct((N, D), jnp.float32),
        ],
    )(agg, x, s, sq, g, be, W, dis)


def _head_body(agg_ref, x_ref, sum_ref, ssq_ref, g_ref, be_ref, w_ref, bh_ref,
               out_ref):
    y = _bn_relu_res(agg_ref, x_ref, sum_ref, ssq_ref, g_ref, be_ref)
    out_ref[...] = jnp.dot(y, w_ref[...], precision=_HI,
                           preferred_element_type=jnp.float32) + bh_ref[...]


def _head(agg, x, s, sq, g, be, W, bh):
    return pl.pallas_call(
        _head_body,
        grid=(G,),
        in_specs=[
            pl.BlockSpec((R, D), lambda i: (i, 0)),
            pl.BlockSpec((R, D), lambda i: (i, 0)),
            pl.BlockSpec((1, D), lambda i: (0, 0)),
            pl.BlockSpec((1, D), lambda i: (0, 0)),
            pl.BlockSpec((1, D), lambda i: (0, 0)),
            pl.BlockSpec((1, D), lambda i: (0, 0)),
            pl.BlockSpec((D, D), lambda i: (0, 0)),
            pl.BlockSpec((1, D), lambda i: (0, 0)),
        ],
        out_specs=pl.BlockSpec((R, D), lambda i: (i, 0)),
        out_shape=jax.ShapeDtypeStruct((N, D), jnp.float32),
    )(agg, x, s, sq, g, be, W, bh)


def kernel(x, edge_index, W1, b1, g1, be1, W2, b2, g2, be2, Wh, bh):
    del b1, b2
    row2d = edge_index[0].reshape(NW * CH, CW)
    col2d = edge_index[1].reshape(NW * CH, CW)
    zeros_nd = jnp.zeros((N, D), jnp.float32)

    dparts = _deg_parts(edge_index[1])
    hs1, dis = _prep(x, W1, dparts.reshape(NW, N, 1))
    s1 = _edge_scatter(hs1, row2d, col2d, zeros_nd)
    agg1, sm1, sq1 = _agg(s1, hs1, dis)
    x2, hs2 = _bnmm(agg1, x, sm1, sq1, g1.reshape(1, D), be1.reshape(1, D),
                    W2, dis)
    s2 = _edge_scatter(hs2, row2d, col2d, zeros_nd)
    agg2, sm2, sq2 = _agg(s2, hs2, dis)
    return _head(agg2, x2, sm2, sq2, g2.reshape(1, D), be2.reshape(1, D),
                 Wh, bh.reshape(1, D))
